# baseline (speedup 1.0000x reference)
import functools

import jax
import jax.numpy as jnp
import numpy as np
from jax import lax
from jax.experimental import pallas as pl
from jax.experimental.pallas import tpu as pltpu
from jax.experimental.pallas import tpu_sc as plsc

N_POINTS = 262144
N_LEVELS = 16
N_FEATS = 2
HASHMAP_SIZE = 1 << 19
MASK = HASHMAP_SIZE - 1
P1 = -1640531535
P2 = 805459861

_SCALES = [np.float32(16.0 * (1.5 ** l) - 1.0) for l in range(N_LEVELS)]
_RES = [int(np.ceil(16.0 * (1.5 ** l) - 1.0)) + 1 for l in range(N_LEVELS)]
_USE_HASH = [(r ** 3) > HASHMAP_SIZE for r in _RES]

NC = 2
NS = 16
NW = NC * NS
PTS_PER_W = N_POINTS // NW
C = 128
NCHUNK = PTS_PER_W // C
G = C // 16


def _body(x_hbm, grid_hbm, out_hbm, xbuf, idxbuf, featbuf, wbuf, outbuf, sem):
    wid = lax.axis_index("s") * NC + lax.axis_index("c")
    base_pt = wid * PTS_PER_W
    pltpu.sync_copy(x_hbm.at[pl.ds(base_pt, PTS_PER_W)], xbuf)

    iota16 = lax.iota(jnp.int32, 16)
    zeros16 = jnp.zeros((16,), jnp.float32)

    def chunk_body(ch, carry):
        off = ch * C

        def do_level(l):
            scale = _SCALES[l]
            lvl_off = l * HASHMAP_SIZE

            def pass1(g, c1):
                pvec = off + g * 16 + iota16
                spos = (g * 16 + iota16) * 4
                xs = [plsc.load_gather(xbuf, [pvec, jnp.full((16,), d, jnp.int32)])
                      for d in range(3)]
                pos = [x * scale + jnp.float32(0.5) for x in xs]
                pi = [q.astype(jnp.int32) for q in pos]
                for d in range(3):
                    wbuf[d, pl.ds(g * 16, 16)] = pos[d] - pi[d].astype(jnp.float32)
                if _USE_HASH[l]:
                    t0 = [pi[0], pi[0] + 1]
                    m1 = pi[1] * P1
                    m2 = pi[2] * P2
                    t1 = [m1, m1 + P1]
                    t2 = [m2, m2 + P2]
                    for c in range(8):
                        h = t0[c & 1] ^ t1[(c >> 1) & 1] ^ t2[(c >> 2) & 1]
                        idx = ((h & MASK) + lvl_off) * 4
                        plsc.store_scatter(
                            idxbuf, [jnp.full((16,), c, jnp.int32), spos], idx)
                else:
                    res = _RES[l]
                    m = res - 1
                    t0 = [jnp.minimum(pi[0], m), jnp.minimum(pi[0] + 1, m)]
                    t1 = [jnp.minimum(pi[1], m) * res,
                          jnp.minimum(pi[1] + 1, m) * res]
                    t2 = [jnp.minimum(pi[2], m) * (res * res),
                          jnp.minimum(pi[2] + 1, m) * (res * res)]
                    for c in range(8):
                        idx = t0[c & 1] + t1[(c >> 1) & 1] + t2[(c >> 2) & 1]
                        idx = (idx + lvl_off) * 4
                        plsc.store_scatter(
                            idxbuf, [jnp.full((16,), c, jnp.int32), spos], idx)
                return c1

            lax.fori_loop(0, G, pass1, 0)

            copies = [pltpu.async_copy(
                grid_hbm.at[idxbuf.at[c]], featbuf.at[c], sem)
                for c in range(8)]
            for cp in copies:
                cp.wait()

            def pass2(g, c2):
                pvec = g * 16 + iota16
                w0 = wbuf[0, pl.ds(g * 16, 16)]
                w1 = wbuf[1, pl.ds(g * 16, 16)]
                w2 = wbuf[2, pl.ds(g * 16, 16)]
                one = jnp.float32(1.0)
                u0, u1, u2 = one - w0, one - w1, one - w2
                a = [u0 * u1, w0 * u1, u0 * w1, w0 * w1]
                wts = [a[0] * u2, a[1] * u2, a[2] * u2, a[3] * u2,
                       a[0] * w2, a[1] * w2, a[2] * w2, a[3] * w2]
                acc0 = zeros16
                acc1 = zeros16
                for c in range(8):
                    cv = jnp.full((16,), c, jnp.int32)
                    f0 = plsc.load_gather(
                        featbuf, [cv, pvec, jnp.zeros((16,), jnp.int32)])
                    f1 = plsc.load_gather(
                        featbuf, [cv, pvec, jnp.ones((16,), jnp.int32)])
                    acc0 = acc0 + wts[c] * f0
                    acc1 = acc1 + wts[c] * f1
                j0, j1 = 2 * l, 2 * l + 1
                outbuf[j0 >> 3, j0 & 7, pl.ds(g * 16, 16)] = acc0
                outbuf[j1 >> 3, j1 & 7, pl.ds(g * 16, 16)] = acc1
                return c2

            lax.fori_loop(0, G, pass2, 0)

        for l in range(N_LEVELS):
            do_level(l)

        t = wid * NCHUNK + ch
        for a4 in range(4):
            pltpu.sync_copy(outbuf.at[a4], out_hbm.at[a4, t])
        return carry

    lax.fori_loop(0, NCHUNK, chunk_body, 0)


_encode_sc = functools.partial(
    pl.kernel,
    mesh=plsc.VectorSubcoreMesh(core_axis_name="c", subcore_axis_name="s"),
    compiler_params=pltpu.CompilerParams(
        needs_layout_passes=False, use_tc_tiling_on_sc=False),
    out_type=jax.ShapeDtypeStruct((4, N_POINTS // C, 8, C), jnp.float32),
    scratch_types=[
        pltpu.VMEM((PTS_PER_W, 3), jnp.float32),
        pltpu.VMEM((8, 4 * C), jnp.int32),
        pltpu.VMEM((8, 4 * C, N_FEATS), jnp.float32),
        pltpu.VMEM((3, C), jnp.float32),
        pltpu.VMEM((4, 8, C), jnp.float32),
        pltpu.SemaphoreType.DMA,
    ],
)(_body)


def kernel(x, grid):
    gflat = jax.lax.optimization_barrier(grid.reshape(-1))
    grid2 = gflat.reshape(N_LEVELS * HASHMAP_SIZE, N_FEATS)
    phys = _encode_sc(x, grid2)
    return phys.transpose(1, 3, 0, 2).reshape(N_POINTS, N_LEVELS * N_FEATS)


# speedup vs baseline: 6.9327x; 6.9327x over previous
import functools

import jax
import jax.numpy as jnp
import numpy as np
from jax import lax
from jax.experimental import pallas as pl
from jax.experimental.pallas import tpu as pltpu
from jax.experimental.pallas import tpu_sc as plsc

N_POINTS = 262144
N_LEVELS = 16
N_FEATS = 2
HASHMAP_SIZE = 1 << 19
MASK = HASHMAP_SIZE - 1
P1 = -1640531535
P2 = 805459861

_SCALES = [np.float32(16.0 * (1.5 ** l) - 1.0) for l in range(N_LEVELS)]
_RES = [int(np.ceil(16.0 * (1.5 ** l) - 1.0)) + 1 for l in range(N_LEVELS)]
_USE_HASH = [(r ** 3) > HASHMAP_SIZE for r in _RES]

NC = 2
NS = 16
NW = NC * NS
PTS_PER_W = N_POINTS // NW
C = 128
NCHUNK = PTS_PER_W // C
G = C // 16


def _body(x_hbm, grid_hbm, out_hbm, xbuf, idxbuf, featbuf, wbuf, outbuf, sem):
    wid = lax.axis_index("s") * NC + lax.axis_index("c")
    base_pt = wid * PTS_PER_W
    pltpu.sync_copy(x_hbm.at[pl.ds(base_pt, PTS_PER_W)], xbuf)

    iota16 = lax.iota(jnp.int32, 16)
    zeros16 = jnp.zeros((16,), jnp.float32)

    def chunk_body(ch, carry):
        off = ch * C

        def do_level(l):
            scale = _SCALES[l]
            lvl_off = l * HASHMAP_SIZE

            def pass1(g, c1):
                pvec = off + g * 16 + iota16
                spos = (g * 16 + iota16) * 4
                xs = [plsc.load_gather(xbuf, [pvec, jnp.full((16,), d, jnp.int32)])
                      for d in range(3)]
                pos = [x * scale + jnp.float32(0.5) for x in xs]
                pi = [q.astype(jnp.int32) for q in pos]
                for d in range(3):
                    wbuf[d, pl.ds(g * 16, 16)] = pos[d] - pi[d].astype(jnp.float32)
                if _USE_HASH[l]:
                    t0 = [pi[0], pi[0] + 1]
                    m1 = pi[1] * P1
                    m2 = pi[2] * P2
                    t1 = [m1, m1 + P1]
                    t2 = [m2, m2 + P2]
                    for c in range(8):
                        h = t0[c & 1] ^ t1[(c >> 1) & 1] ^ t2[(c >> 2) & 1]
                        idx = ((h & MASK) + lvl_off) * 4
                        plsc.store_scatter(
                            idxbuf, [jnp.full((16,), c, jnp.int32), spos], idx)
                else:
                    res = _RES[l]
                    m = res - 1
                    t0 = [jnp.minimum(pi[0], m), jnp.minimum(pi[0] + 1, m)]
                    t1 = [jnp.minimum(pi[1], m) * res,
                          jnp.minimum(pi[1] + 1, m) * res]
                    t2 = [jnp.minimum(pi[2], m) * (res * res),
                          jnp.minimum(pi[2] + 1, m) * (res * res)]
                    for c in range(8):
                        idx = t0[c & 1] + t1[(c >> 1) & 1] + t2[(c >> 2) & 1]
                        idx = (idx + lvl_off) * 4
                        plsc.store_scatter(
                            idxbuf, [jnp.full((16,), c, jnp.int32), spos], idx)
                return c1

            lax.fori_loop(0, G, pass1, 0)

            copies = [pltpu.async_copy(
                grid_hbm.at[idxbuf.at[c]], featbuf.at[c], sem)
                for c in range(8)]
            for cp in copies:
                cp.wait()

            def pass2(g, c2):
                pvec = g * 16 + iota16
                w0 = wbuf[0, pl.ds(g * 16, 16)]
                w1 = wbuf[1, pl.ds(g * 16, 16)]
                w2 = wbuf[2, pl.ds(g * 16, 16)]
                one = jnp.float32(1.0)
                u0, u1, u2 = one - w0, one - w1, one - w2
                a = [u0 * u1, w0 * u1, u0 * w1, w0 * w1]
                wts = [a[0] * u2, a[1] * u2, a[2] * u2, a[3] * u2,
                       a[0] * w2, a[1] * w2, a[2] * w2, a[3] * w2]
                acc0 = zeros16
                acc1 = zeros16
                for c in range(8):
                    cv = jnp.full((16,), c, jnp.int32)
                    f0 = plsc.load_gather(
                        featbuf, [cv, pvec, jnp.zeros((16,), jnp.int32)])
                    f1 = plsc.load_gather(
                        featbuf, [cv, pvec, jnp.ones((16,), jnp.int32)])
                    acc0 = acc0 + wts[c] * f0
                    acc1 = acc1 + wts[c] * f1
                j0, j1 = 2 * l, 2 * l + 1
                outbuf[j0 >> 3, j0 & 7, pl.ds(g * 16, 16)] = acc0
                outbuf[j1 >> 3, j1 & 7, pl.ds(g * 16, 16)] = acc1
                return c2

            lax.fori_loop(0, G, pass2, 0)

        for l in range(N_LEVELS):
            do_level(l)

        t = wid * NCHUNK + ch
        for a4 in range(4):
            pltpu.sync_copy(outbuf.at[a4], out_hbm.at[a4, t])
        return carry

    lax.fori_loop(0, NCHUNK, chunk_body, 0)


_encode_sc = functools.partial(
    pl.kernel,
    mesh=plsc.VectorSubcoreMesh(core_axis_name="c", subcore_axis_name="s"),
    compiler_params=pltpu.CompilerParams(
        needs_layout_passes=False, use_tc_tiling_on_sc=False),
    out_type=jax.ShapeDtypeStruct((4, N_POINTS // C, 8, C), jnp.float32),
    scratch_types=[
        pltpu.VMEM((PTS_PER_W, 3), jnp.float32),
        pltpu.VMEM((8, 4 * C), jnp.int32),
        pltpu.VMEM((8, 4 * C, N_FEATS), jnp.float32),
        pltpu.VMEM((3, C), jnp.float32),
        pltpu.VMEM((4, 8, C), jnp.float32),
        pltpu.SemaphoreType.DMA,
    ],
)(_body)


def kernel(x, grid):
    grid2 = grid.reshape(N_LEVELS * HASHMAP_SIZE, N_FEATS)
    phys = _encode_sc(x, grid2)
    return phys.transpose(1, 3, 0, 2).reshape(N_POINTS, N_LEVELS * N_FEATS)


# speedup vs baseline: 23.2425x; 3.3526x over previous
import functools

import jax
import jax.numpy as jnp
import numpy as np
from jax import lax
from jax.experimental import pallas as pl
from jax.experimental.pallas import tpu as pltpu
from jax.experimental.pallas import tpu_sc as plsc

N_POINTS = 262144
N_LEVELS = 16
N_FEATS = 2
HASHMAP_SIZE = 1 << 19
MASK = HASHMAP_SIZE - 1
P1 = -1640531535
P2 = 805459861

_SCALES = [np.float32(16.0 * (1.5 ** l) - 1.0) for l in range(N_LEVELS)]
_RES = [int(np.ceil(16.0 * (1.5 ** l) - 1.0)) + 1 for l in range(N_LEVELS)]
_USE_HASH = [(r ** 3) > HASHMAP_SIZE for r in _RES]

NC = 2
NS = 16
NW = NC * NS
PTS_PER_W = N_POINTS // NW
C = 128
NCHUNK = PTS_PER_W // C
G = C // 16


def _body(x_hbm, grid_hbm, out_hbm, xbuf, idxbuf, featbuf, wbuf, pbuf, outbuf,
          sem):
    wid = lax.axis_index("s") * NC + lax.axis_index("c")
    base_pt = wid * PTS_PER_W

    iota16 = lax.iota(jnp.int32, 16)
    zeros16 = jnp.zeros((16,), jnp.float32)

    def chunk_body(ch, carry):
        off = ch * C
        pltpu.sync_copy(x_hbm.at[pl.ds(base_pt + off, C)], xbuf)

        def do_level(l):
            scale = _SCALES[l]
            lvl_off = l * (HASHMAP_SIZE * 2)

            def pass1(g, c1):
                pvec = g * 16 + iota16
                spos = (g * 16 + iota16) * 4
                xs = [plsc.load_gather(xbuf, [pvec, jnp.full((16,), d, jnp.int32)])
                      for d in range(3)]
                pos = [x * scale + jnp.float32(0.5) for x in xs]
                pi = [q.astype(jnp.int32) for q in pos]
                for d in range(3):
                    wbuf[d, pl.ds(g * 16, 16)] = pos[d] - pi[d].astype(jnp.float32)
                def emit(c, idx):
                    pf0 = lvl_off + ((idx >> 7) << 8) + (idx & 127)
                    v0 = (pf0 >> 1) << 2
                    plsc.store_scatter(
                        idxbuf, [jnp.full((16,), 2 * c, jnp.int32), spos], v0)
                    plsc.store_scatter(
                        idxbuf, [jnp.full((16,), 2 * c + 1, jnp.int32), spos],
                        v0 + 256)
                    pbuf[c, pl.ds(g * 16, 16)] = idx & 1

                if _USE_HASH[l]:
                    t0 = [pi[0], pi[0] + 1]
                    m1 = pi[1] * P1
                    m2 = pi[2] * P2
                    t1 = [m1, m1 + P1]
                    t2 = [m2, m2 + P2]
                    for c in range(8):
                        h = t0[c & 1] ^ t1[(c >> 1) & 1] ^ t2[(c >> 2) & 1]
                        emit(c, h & MASK)
                else:
                    res = _RES[l]
                    m = res - 1
                    t0 = [jnp.minimum(pi[0], m), jnp.minimum(pi[0] + 1, m)]
                    t1 = [jnp.minimum(pi[1], m) * res,
                          jnp.minimum(pi[1] + 1, m) * res]
                    t2 = [jnp.minimum(pi[2], m) * (res * res),
                          jnp.minimum(pi[2] + 1, m) * (res * res)]
                    for c in range(8):
                        emit(c, t0[c & 1] + t1[(c >> 1) & 1] + t2[(c >> 2) & 1])
                return c1

            lax.fori_loop(0, G, pass1, 0)

            copies = [pltpu.async_copy(
                grid_hbm.at[idxbuf.at[c]], featbuf.at[c], sem)
                for c in range(16)]
            for cp in copies:
                cp.wait()

            def pass2(g, c2):
                pvec = g * 16 + iota16
                w0 = wbuf[0, pl.ds(g * 16, 16)]
                w1 = wbuf[1, pl.ds(g * 16, 16)]
                w2 = wbuf[2, pl.ds(g * 16, 16)]
                one = jnp.float32(1.0)
                u0, u1, u2 = one - w0, one - w1, one - w2
                a = [u0 * u1, w0 * u1, u0 * w1, w0 * w1]
                wts = [a[0] * u2, a[1] * u2, a[2] * u2, a[3] * u2,
                       a[0] * w2, a[1] * w2, a[2] * w2, a[3] * w2]
                acc0 = zeros16
                acc1 = zeros16
                for c in range(8):
                    par = pbuf[c, pl.ds(g * 16, 16)]
                    f0 = plsc.load_gather(
                        featbuf, [jnp.full((16,), 2 * c, jnp.int32), pvec, par])
                    f1 = plsc.load_gather(
                        featbuf,
                        [jnp.full((16,), 2 * c + 1, jnp.int32), pvec, par])
                    acc0 = acc0 + wts[c] * f0
                    acc1 = acc1 + wts[c] * f1
                plsc.store_scatter(
                    outbuf, [pvec, jnp.full((16,), 2 * l, jnp.int32)], acc0)
                plsc.store_scatter(
                    outbuf, [pvec, jnp.full((16,), 2 * l + 1, jnp.int32)], acc1)
                return c2

            lax.fori_loop(0, G, pass2, 0)

        for l in range(N_LEVELS):
            do_level(l)

        pltpu.sync_copy(outbuf, out_hbm.at[pl.ds(base_pt + off, C)])
        return carry

    lax.fori_loop(0, NCHUNK, chunk_body, 0)


_encode_sc = functools.partial(
    pl.kernel,
    mesh=plsc.VectorSubcoreMesh(core_axis_name="c", subcore_axis_name="s"),
    compiler_params=pltpu.CompilerParams(
        needs_layout_passes=False, use_tc_tiling_on_sc=False),
    out_type=jax.ShapeDtypeStruct((N_POINTS, N_LEVELS * N_FEATS), jnp.float32),
    scratch_types=[
        pltpu.VMEM((C, 3), jnp.float32),
        pltpu.VMEM((16, 4 * C), jnp.int32),
        pltpu.VMEM((16, 4 * C, N_FEATS), jnp.float32),
        pltpu.VMEM((3, C), jnp.float32),
        pltpu.VMEM((8, C), jnp.int32),
        pltpu.VMEM((C, N_LEVELS * N_FEATS), jnp.float32),
        pltpu.SemaphoreType.DMA,
    ],
)(_body)


def kernel(x, grid):
    gview = jnp.transpose(
        grid.reshape(N_LEVELS, 4096, 128, N_FEATS), (0, 1, 3, 2))
    grid2 = gview.reshape(N_LEVELS * HASHMAP_SIZE, N_FEATS)
    return _encode_sc(x, grid2)


# speedup vs baseline: 71.3225x; 3.0686x over previous
import functools

import jax
import jax.numpy as jnp
import numpy as np
from jax import lax
from jax.experimental import pallas as pl
from jax.experimental.pallas import tpu as pltpu
from jax.experimental.pallas import tpu_sc as plsc

N_POINTS = 262144
N_LEVELS = 16
N_FEATS = 2
HASHMAP_SIZE = 1 << 19
MASK = HASHMAP_SIZE - 1
P1 = -1640531535
P2 = 805459861

_SCALES = [np.float32(16.0 * (1.5 ** l) - 1.0) for l in range(N_LEVELS)]
_RES = [int(np.ceil(16.0 * (1.5 ** l) - 1.0)) + 1 for l in range(N_LEVELS)]
_USE_HASH = [(r ** 3) > HASHMAP_SIZE for r in _RES]

NC = 2
NS = 16
NW = NC * NS
PTS_PER_W = N_POINTS // NW
C = 128
NCHUNK = PTS_PER_W // C
G = C // 16
TB = 64
NTCOL = HASHMAP_SIZE // 128


def _body(x_hbm, gview_hbm, out_hbm, gscratch_hbm, xbuf, cbuf, obuf, idxbuf,
          featbuf, wbuf, outbuf, sem):
    ci = lax.axis_index("c")
    si = lax.axis_index("s")
    wid = si * NC + ci
    base_pt = wid * PTS_PER_W

    iota16 = lax.iota(jnp.int32, 16)
    zeros16 = jnp.zeros((16,), jnp.float32)

    def blk_body(b, cA):
        t0 = b * TB
        pltpu.sync_copy(gview_hbm.at[si, pl.ds(t0, TB)], cbuf)

        def tt_body(tt, cB):
            jloc = tt * 128
            for cg in range(8):
                pv = jloc + cg * 16 + iota16
                f0 = cbuf[tt, 0, pl.ds(cg * 16, 16)]
                f1 = cbuf[tt, 1, pl.ds(cg * 16, 16)]
                plsc.store_scatter(obuf, [pv, jnp.zeros((16,), jnp.int32)], f0)
                plsc.store_scatter(obuf, [pv, jnp.ones((16,), jnp.int32)], f1)
            return cB

        lax.fori_loop(0, TB, tt_body, 0)
        pltpu.sync_copy(
            obuf, gscratch_hbm.at[ci, pl.ds(si * HASHMAP_SIZE + t0 * 128,
                                            TB * 128)])
        return cA

    lax.fori_loop(0, NTCOL // TB, blk_body, 0)
    plsc.subcore_barrier()

    def chunk_body(ch, carry):
        off = ch * C
        pltpu.sync_copy(x_hbm.at[pl.ds(base_pt + off, C)], xbuf)

        def do_level(l):
            scale = _SCALES[l]
            lvl_off = l * HASHMAP_SIZE

            def pass1(g, c1):
                pvec = g * 16 + iota16
                spos = (g * 16 + iota16) * 4
                xs = [plsc.load_gather(xbuf, [pvec, jnp.full((16,), d, jnp.int32)])
                      for d in range(3)]
                pos = [x * scale + jnp.float32(0.5) for x in xs]
                pi = [q.astype(jnp.int32) for q in pos]
                for d in range(3):
                    wbuf[d, pl.ds(g * 16, 16)] = pos[d] - pi[d].astype(jnp.float32)
                if _USE_HASH[l]:
                    t0 = [pi[0], pi[0] + 1]
                    m1 = pi[1] * P1
                    m2 = pi[2] * P2
                    t1 = [m1, m1 + P1]
                    t2 = [m2, m2 + P2]
                    for c in range(8):
                        h = t0[c & 1] ^ t1[(c >> 1) & 1] ^ t2[(c >> 2) & 1]
                        idx = ((h & MASK) + lvl_off) * 4
                        plsc.store_scatter(
                            idxbuf, [jnp.full((16,), c, jnp.int32), spos], idx)
                else:
                    res = _RES[l]
                    m = res - 1
                    t0 = [jnp.minimum(pi[0], m), jnp.minimum(pi[0] + 1, m)]
                    t1 = [jnp.minimum(pi[1], m) * res,
                          jnp.minimum(pi[1] + 1, m) * res]
                    t2 = [jnp.minimum(pi[2], m) * (res * res),
                          jnp.minimum(pi[2] + 1, m) * (res * res)]
                    for c in range(8):
                        idx = t0[c & 1] + t1[(c >> 1) & 1] + t2[(c >> 2) & 1]
                        idx = (idx + lvl_off) * 4
                        plsc.store_scatter(
                            idxbuf, [jnp.full((16,), c, jnp.int32), spos], idx)
                return c1

            lax.fori_loop(0, G, pass1, 0)

            copies = [pltpu.async_copy(
                gscratch_hbm.at[ci].at[idxbuf.at[c]], featbuf.at[c], sem)
                for c in range(8)]
            for cp in copies:
                cp.wait()

            def pass2(g, c2):
                pvec = g * 16 + iota16
                w0 = wbuf[0, pl.ds(g * 16, 16)]
                w1 = wbuf[1, pl.ds(g * 16, 16)]
                w2 = wbuf[2, pl.ds(g * 16, 16)]
                one = jnp.float32(1.0)
                u0, u1, u2 = one - w0, one - w1, one - w2
                a = [u0 * u1, w0 * u1, u0 * w1, w0 * w1]
                wts = [a[0] * u2, a[1] * u2, a[2] * u2, a[3] * u2,
                       a[0] * w2, a[1] * w2, a[2] * w2, a[3] * w2]
                acc0 = zeros16
                acc1 = zeros16
                for c in range(8):
                    cv = jnp.full((16,), c, jnp.int32)
                    f0 = plsc.load_gather(
                        featbuf, [cv, pvec, jnp.zeros((16,), jnp.int32)])
                    f1 = plsc.load_gather(
                        featbuf, [cv, pvec, jnp.ones((16,), jnp.int32)])
                    acc0 = acc0 + wts[c] * f0
                    acc1 = acc1 + wts[c] * f1
                plsc.store_scatter(
                    outbuf, [pvec, jnp.full((16,), 2 * l, jnp.int32)], acc0)
                plsc.store_scatter(
                    outbuf, [pvec, jnp.full((16,), 2 * l + 1, jnp.int32)], acc1)
                return c2

            lax.fori_loop(0, G, pass2, 0)

        for l in range(N_LEVELS):
            do_level(l)

        pltpu.sync_copy(outbuf, out_hbm.at[pl.ds(base_pt + off, C)])
        return carry

    lax.fori_loop(0, NCHUNK, chunk_body, 0)


_encode_sc = functools.partial(
    pl.kernel,
    mesh=plsc.VectorSubcoreMesh(core_axis_name="c", subcore_axis_name="s"),
    compiler_params=pltpu.CompilerParams(
        needs_layout_passes=False, use_tc_tiling_on_sc=False),
    out_type=(
        jax.ShapeDtypeStruct((N_POINTS, N_LEVELS * N_FEATS), jnp.float32),
        jax.ShapeDtypeStruct((NC, N_LEVELS * HASHMAP_SIZE, N_FEATS),
                             jnp.float32),
    ),
    scratch_types=[
        pltpu.VMEM((C, 3), jnp.float32),
        pltpu.VMEM((TB, N_FEATS, 128), jnp.float32),
        pltpu.VMEM((TB * 128, N_FEATS), jnp.float32),
        pltpu.VMEM((8, 4 * C), jnp.int32),
        pltpu.VMEM((8, 4 * C, N_FEATS), jnp.float32),
        pltpu.VMEM((3, C), jnp.float32),
        pltpu.VMEM((C, N_LEVELS * N_FEATS), jnp.float32),
        pltpu.SemaphoreType.DMA,
    ],
)(_body)


def kernel(x, grid):
    gview = jnp.transpose(
        grid.reshape(N_LEVELS, NTCOL, 128, N_FEATS), (0, 1, 3, 2))
    out, _ = _encode_sc(x, gview)
    return out


# speedup vs baseline: 73.3170x; 1.0280x over previous
import functools

import jax
import jax.numpy as jnp
import numpy as np
from jax import lax
from jax.experimental import pallas as pl
from jax.experimental.pallas import tpu as pltpu
from jax.experimental.pallas import tpu_sc as plsc

N_POINTS = 262144
N_LEVELS = 16
N_FEATS = 2
HASHMAP_SIZE = 1 << 19
MASK = HASHMAP_SIZE - 1
P1 = -1640531535
P2 = 805459861

_SCALES = [np.float32(16.0 * (1.5 ** l) - 1.0) for l in range(N_LEVELS)]
_RES = [int(np.ceil(16.0 * (1.5 ** l) - 1.0)) + 1 for l in range(N_LEVELS)]
_USE_HASH = [(r ** 3) > HASHMAP_SIZE for r in _RES]

NC = 2
NS = 16
NW = NC * NS
PTS_PER_W = N_POINTS // NW
C = 128
NCHUNK = PTS_PER_W // C
G = C // 16
TB = 64
NTCOL = HASHMAP_SIZE // 128


def _body(x_hbm, gview_hbm, out_hbm, gscratch_hbm, xbuf, cbuf, obuf, idxbuf,
          featbuf, wbuf, outbuf, sem0, sem1):
    sems = (sem0, sem1)
    ci = lax.axis_index("c")
    si = lax.axis_index("s")
    wid = si * NC + ci
    base_pt = wid * PTS_PER_W

    iota16 = lax.iota(jnp.int32, 16)
    zeros16 = jnp.zeros((16,), jnp.float32)

    def blk_body(b, cA):
        t0 = b * TB
        pltpu.sync_copy(gview_hbm.at[si, pl.ds(t0, TB)], cbuf)

        def tt_body(tt, cB):
            jloc = tt * 128
            for cg in range(8):
                pv = jloc + cg * 16 + iota16
                f0 = cbuf[tt, 0, pl.ds(cg * 16, 16)]
                f1 = cbuf[tt, 1, pl.ds(cg * 16, 16)]
                plsc.store_scatter(obuf, [pv, jnp.zeros((16,), jnp.int32)], f0)
                plsc.store_scatter(obuf, [pv, jnp.ones((16,), jnp.int32)], f1)
            return cB

        lax.fori_loop(0, TB, tt_body, 0)
        pltpu.sync_copy(
            obuf, gscratch_hbm.at[ci, pl.ds(si * HASHMAP_SIZE + t0 * 128,
                                            TB * 128)])
        return cA

    lax.fori_loop(0, NTCOL // TB, blk_body, 0)
    plsc.subcore_barrier()

    def chunk_body(ch, carry):
        off = ch * C
        pltpu.sync_copy(x_hbm.at[pl.ds(base_pt + off, C)], xbuf)

        def make_pass1(l, par):
            scale = _SCALES[l]
            lvl_off = l * HASHMAP_SIZE

            def pass1(g, c1):
                pvec = g * 16 + iota16
                spos = (g * 16 + iota16) * 4
                xs = [plsc.load_gather(xbuf, [pvec, jnp.full((16,), d, jnp.int32)])
                      for d in range(3)]
                pos = [x * scale + jnp.float32(0.5) for x in xs]
                pi = [q.astype(jnp.int32) for q in pos]
                for d in range(3):
                    wbuf[par, d, pl.ds(g * 16, 16)] = (
                        pos[d] - pi[d].astype(jnp.float32))
                if _USE_HASH[l]:
                    t0 = [pi[0], pi[0] + 1]
                    m1 = pi[1] * P1
                    m2 = pi[2] * P2
                    t1 = [m1, m1 + P1]
                    t2 = [m2, m2 + P2]
                    for c in range(8):
                        h = t0[c & 1] ^ t1[(c >> 1) & 1] ^ t2[(c >> 2) & 1]
                        idx = ((h & MASK) + lvl_off) * 4
                        plsc.store_scatter(
                            idxbuf.at[par],
                            [jnp.full((16,), c, jnp.int32), spos], idx)
                else:
                    res = _RES[l]
                    m = res - 1
                    t0 = [jnp.minimum(pi[0], m), jnp.minimum(pi[0] + 1, m)]
                    t1 = [jnp.minimum(pi[1], m) * res,
                          jnp.minimum(pi[1] + 1, m) * res]
                    t2 = [jnp.minimum(pi[2], m) * (res * res),
                          jnp.minimum(pi[2] + 1, m) * (res * res)]
                    for c in range(8):
                        idx = t0[c & 1] + t1[(c >> 1) & 1] + t2[(c >> 2) & 1]
                        idx = (idx + lvl_off) * 4
                        plsc.store_scatter(
                            idxbuf.at[par],
                            [jnp.full((16,), c, jnp.int32), spos], idx)
                return c1

            return pass1

        def fire(l):
            par = l % 2
            return [pltpu.async_copy(
                gscratch_hbm.at[ci].at[idxbuf.at[par, c]],
                featbuf.at[c], sems[par])
                for c in range(8)]

        def make_pass2(l, par):

            def pass2(g, c2):
                pvec = g * 16 + iota16
                w0 = wbuf[par, 0, pl.ds(g * 16, 16)]
                w1 = wbuf[par, 1, pl.ds(g * 16, 16)]
                w2 = wbuf[par, 2, pl.ds(g * 16, 16)]
                one = jnp.float32(1.0)
                u0, u1, u2 = one - w0, one - w1, one - w2
                a = [u0 * u1, w0 * u1, u0 * w1, w0 * w1]
                wts = [a[0] * u2, a[1] * u2, a[2] * u2, a[3] * u2,
                       a[0] * w2, a[1] * w2, a[2] * w2, a[3] * w2]
                acc0 = zeros16
                acc1 = zeros16
                for c in range(8):
                    cv = jnp.full((16,), c, jnp.int32)
                    f0 = plsc.load_gather(
                        featbuf, [cv, pvec, jnp.zeros((16,), jnp.int32)])
                    f1 = plsc.load_gather(
                        featbuf, [cv, pvec, jnp.ones((16,), jnp.int32)])
                    acc0 = acc0 + wts[c] * f0
                    acc1 = acc1 + wts[c] * f1
                plsc.store_scatter(
                    outbuf, [pvec, jnp.full((16,), 2 * l, jnp.int32)], acc0)
                plsc.store_scatter(
                    outbuf, [pvec, jnp.full((16,), 2 * l + 1, jnp.int32)], acc1)
                return c2

            return pass2

        lax.fori_loop(0, G, make_pass1(0, 0), 0)
        hs = fire(0)
        for l in range(N_LEVELS):
            if l + 1 < N_LEVELS:
                lax.fori_loop(0, G, make_pass1(l + 1, (l + 1) % 2), 0)
            for cp in hs:
                cp.wait()
            lax.fori_loop(0, G, make_pass2(l, l % 2), 0)
            hs = fire(l + 1) if l + 1 < N_LEVELS else None

        pltpu.sync_copy(outbuf, out_hbm.at[pl.ds(base_pt + off, C)])
        return carry

    lax.fori_loop(0, NCHUNK, chunk_body, 0)


_encode_sc = functools.partial(
    pl.kernel,
    mesh=plsc.VectorSubcoreMesh(core_axis_name="c", subcore_axis_name="s"),
    compiler_params=pltpu.CompilerParams(
        needs_layout_passes=False, use_tc_tiling_on_sc=False),
    out_type=(
        jax.ShapeDtypeStruct((N_POINTS, N_LEVELS * N_FEATS), jnp.float32),
        jax.ShapeDtypeStruct((NC, N_LEVELS * HASHMAP_SIZE, N_FEATS),
                             jnp.float32),
    ),
    scratch_types=[
        pltpu.VMEM((C, 3), jnp.float32),
        pltpu.VMEM((TB, N_FEATS, 128), jnp.float32),
        pltpu.VMEM((TB * 128, N_FEATS), jnp.float32),
        pltpu.VMEM((2, 8, 4 * C), jnp.int32),
        pltpu.VMEM((8, 4 * C, N_FEATS), jnp.float32),
        pltpu.VMEM((2, 3, C), jnp.float32),
        pltpu.VMEM((C, N_LEVELS * N_FEATS), jnp.float32),
        pltpu.SemaphoreType.DMA,
        pltpu.SemaphoreType.DMA,
    ],
)(_body)


def kernel(x, grid):
    gview = jnp.transpose(
        grid.reshape(N_LEVELS, NTCOL, 128, N_FEATS), (0, 1, 3, 2))
    out, _ = _encode_sc(x, gview)
    return out
